# baseline (speedup 1.0000x reference)
import functools

import jax
import jax.numpy as jnp
import numpy as np
from jax import lax
from jax.experimental import pallas as pl
from jax.experimental.pallas import tpu as pltpu
from jax.experimental.pallas import tpu_sc as plsc

_S = 512
_D = 64
_B = 2
_OFF = 2
_NTRI = (_S - _OFF) * (_S - _OFF + 1) // 2
_NOUT = _B * _NTRI

_CHUNK = 128
_NW = 32
_CH_PER_W = 64
_NCH_FULL = _NOUT // _CHUNK
_NCH_VALID = _NCH_FULL + 1
_LAST_BASE = _NOUT - _CHUNK
_NBUF = 4


def _build_index_chunks() -> np.ndarray:
    r, c = np.triu_indices(_S, _OFF)
    idx0 = (r + _S * c).astype(np.int32)
    flat = np.concatenate([idx0, idx0 + _S * _S])
    chunks = np.zeros((_NW * _CH_PER_W, _CHUNK), np.int32)
    chunks[:_NCH_FULL] = flat[: _NCH_FULL * _CHUNK].reshape(_NCH_FULL, _CHUNK)
    chunks[_NCH_FULL] = flat[_LAST_BASE:]
    return chunks.reshape(_NW, _CH_PER_W, _CHUNK)


_IDX_CHUNKS = _build_index_chunks()


@functools.cache
def _make_triu_gather():
    mesh = plsc.VectorSubcoreMesh(
        core_axis_name="c", subcore_axis_name="s", num_cores=2, num_subcores=16
    )
    return functools.partial(
        pl.kernel,
        out_type=jax.ShapeDtypeStruct((_NOUT, _D), jnp.float32),
        mesh=mesh,
        compiler_params=pltpu.CompilerParams(use_tc_tiling_on_sc=False),
        scratch_types=[
            pltpu.VMEM((_CH_PER_W, _CHUNK), jnp.int32),
            [pltpu.VMEM((_CHUNK, _D), jnp.float32)] * _NBUF,
            [pltpu.SemaphoreType.DMA] * _NBUF,
            [pltpu.SemaphoreType.DMA] * _NBUF,
        ],
    )(_triu_gather)


def _triu_gather(table_hbm, idx_hbm, out_hbm, idx_v, bufs, gsems, wsems):
    wid = lax.axis_index("s") * 2 + lax.axis_index("c")
    c0 = wid * _CH_PER_W
    pltpu.sync_copy(idx_hbm.at[wid], idx_v)

    def chunk_ok(j):
        return jnp.logical_and(j < _CH_PER_W, c0 + j < _NCH_VALID)

    def gather_start(j, s):
        @pl.when(chunk_ok(j))
        def _():
            pltpu.async_copy(table_hbm.at[idx_v.at[j]], bufs[s], gsems[s])

    def write_start(j, s):
        @pl.when(chunk_ok(j))
        def _():
            pltpu.make_async_copy(table_hbm.at[idx_v.at[j]], bufs[s],
                                  gsems[s]).wait()
            base = lax.min((c0 + j) * _CHUNK, _LAST_BASE)
            pltpu.async_copy(bufs[s], out_hbm.at[pl.ds(base, _CHUNK)],
                             wsems[s])

    def write_wait(j, s):
        @pl.when(chunk_ok(j))
        def _():
            base = lax.min((c0 + j) * _CHUNK, _LAST_BASE)
            pltpu.make_async_copy(bufs[s], out_hbm.at[pl.ds(base, _CHUNK)],
                                  wsems[s]).wait()

    for s in range(_NBUF):
        gather_start(s, s)

    def step(i, carry):
        j = i * _NBUF
        for s in range(_NBUF):
            write_start(j + s, s)
        for s in range(_NBUF):
            write_wait(j + s, s)
            gather_start(j + _NBUF + s, s)
        return carry

    lax.fori_loop(0, _CH_PER_W // _NBUF, step, 0)


def kernel(inputs):
    table = inputs.reshape(_B * _S * _S, _D)
    out = _make_triu_gather()(table, jnp.asarray(_IDX_CHUNKS))
    return out.reshape(_B, _NTRI, _D)
